# Initial kernel scaffold; baseline (speedup 1.0000x reference)
import jax
import jax.numpy as jnp
from jax.experimental import pallas as pl


def kernel(x, gate_w, gate_bias, shared_gate_w, shared_down_w, up_w, down_w):
    raise NotImplementedError("write your pallas kernel here")


# speedup vs baseline: 2.9442x; 2.9442x over previous
import functools

import jax
import jax.numpy as jnp
from jax import lax
from jax.experimental import pallas as pl
from jax.experimental.pallas import tpu as pltpu
from jax.experimental.pallas import tpu_sc as plsc

B, T, C = 1, 2048, 2048
E, TOPK = 16, 2
H = 1024
HS = 2048
N = B * T
NA = N * TOPK
TM = 128
W = 48
P = W * TM

NC, NS = 2, 16
NW = NC * NS

@functools.lru_cache(maxsize=None)
def _sc_mesh():
    return plsc.VectorSubcoreMesh(core_axis_name="c", subcore_axis_name="s")


def _router_body(x_ref, gw_ref, gb_ref, idx_ref, w_ref):
    s = jax.nn.sigmoid(
        jnp.dot(x_ref[...], gw_ref[...], preferred_element_type=jnp.float32))
    sb = s + gb_ref[...]
    tm = s.shape[0]
    iota = lax.broadcasted_iota(jnp.int32, (tm, E), 1)
    m1 = jnp.max(sb, axis=1, keepdims=True)
    i1 = jnp.min(jnp.where(sb == m1, iota, E), axis=1, keepdims=True)
    sb2 = jnp.where(iota == i1, -jnp.inf, sb)
    m2 = jnp.max(sb2, axis=1, keepdims=True)
    i2 = jnp.min(jnp.where(sb2 == m2, iota, E), axis=1, keepdims=True)
    w1 = jnp.sum(jnp.where(iota == i1, s, 0.0), axis=1, keepdims=True)
    w2 = jnp.sum(jnp.where(iota == i2, s, 0.0), axis=1, keepdims=True)
    den = w1 + w2
    idx_ref[...] = jnp.where(iota == 0, i1, jnp.where(iota == 1, i2, 0))
    w_ref[...] = jnp.where(iota == 0, w1 / den,
                           jnp.where(iota == 1, w2 / den, 0.0))


def _router(x_flat, gate_w, gate_bias):
    tm = 256
    return pl.pallas_call(
        _router_body,
        grid=(N // tm,),
        in_specs=[
            pl.BlockSpec((tm, C), lambda t: (t, 0)),
            pl.BlockSpec((C, E), lambda t: (0, 0)),
            pl.BlockSpec((1, E), lambda t: (0, 0)),
        ],
        out_specs=[
            pl.BlockSpec((tm, E), lambda t: (t, 0)),
            pl.BlockSpec((tm, E), lambda t: (t, 0)),
        ],
        out_shape=[
            jax.ShapeDtypeStruct((N, E), jnp.int32),
            jax.ShapeDtypeStruct((N, E), jnp.float32),
        ],
    )(x_flat, gate_w, gate_bias.reshape(1, E))


_D_RPW = P // NW
_D_CH = 32
_D_NCH = _D_RPW // _D_CH


def _dispatch_body(x_hbm, tok_hbm, out_hbm, idx_v, buf, sem):
    wid = lax.axis_index("s") * NC + lax.axis_index("c")
    base = wid * _D_RPW
    pltpu.sync_copy(tok_hbm.at[pl.ds(base, _D_RPW)], idx_v)
    for ci in range(_D_NCH):
        pltpu.async_copy(x_hbm.at[idx_v.at[pl.ds(ci * _D_CH, _D_CH)]],
                         buf, sem).wait()
        pltpu.sync_copy(buf, out_hbm.at[pl.ds(base + ci * _D_CH, _D_CH)])


@functools.lru_cache(maxsize=None)
def _dispatch_kernel():
    return pl.kernel(
        _dispatch_body,
        out_type=jax.ShapeDtypeStruct((P, C), jnp.float32),
        mesh=_sc_mesh(),
        scratch_types=[
            pltpu.VMEM((_D_RPW,), jnp.int32),
            pltpu.VMEM((_D_CH, C), jnp.float32),
            pltpu.SemaphoreType.DMA,
        ],
    )


def _dispatch(x_flat, tok1d):
    return _dispatch_kernel()(x_flat, tok1d)


def _gemm_body(e_ref, v_ref, x_ref, up_ref, dn_ref, w_ref, o_ref):
    i = pl.program_id(0)

    @pl.when(v_ref[i] == 1)
    def _():
        yg = jnp.dot(x_ref[...], up_ref[0],
                     preferred_element_type=jnp.float32)
        gv = yg[:, :H]
        uv = yg[:, H:]
        h = gv * jax.nn.sigmoid(gv) * uv
        oe = jnp.dot(h, dn_ref[0], preferred_element_type=jnp.float32)
        o_ref[...] = oe * w_ref[:, 0:1]


def _grouped_gemm(x_perm, up_w, down_w, wpad2d, e_for_tile, valid):
    grid_spec = pltpu.PrefetchScalarGridSpec(
        num_scalar_prefetch=2,
        grid=(W,),
        in_specs=[
            pl.BlockSpec((TM, C), lambda i, e, v: (i, 0)),
            pl.BlockSpec((1, C, 2 * H), lambda i, e, v: (e[i], 0, 0)),
            pl.BlockSpec((1, H, C), lambda i, e, v: (e[i], 0, 0)),
            pl.BlockSpec((TM, 128), lambda i, e, v: (i, 0)),
        ],
        out_specs=pl.BlockSpec((TM, C), lambda i, e, v: (i, 0)),
    )
    return pl.pallas_call(
        _gemm_body,
        grid_spec=grid_spec,
        out_shape=jax.ShapeDtypeStruct((P, C), jnp.float32),
    )(e_for_tile, valid, x_perm, up_w, down_w, wpad2d)


_C_RPW = N // NW
_C_CH = 16
_C_NCH = _C_RPW // _C_CH


def _combine_body(src_hbm, p0_hbm, p1_hbm, g0_hbm, g1_hbm, i0_v, i1_v, buf,
                  sem):
    wid = lax.axis_index("s") * NC + lax.axis_index("c")
    base = wid * _C_RPW
    pltpu.sync_copy(p0_hbm.at[pl.ds(base, _C_RPW)], i0_v)
    pltpu.sync_copy(p1_hbm.at[pl.ds(base, _C_RPW)], i1_v)
    for ci in range(_C_NCH):
        sl = pl.ds(ci * _C_CH, _C_CH)
        pltpu.async_copy(src_hbm.at[i0_v.at[sl]], buf, sem).wait()
        pltpu.sync_copy(buf, g0_hbm.at[pl.ds(base + ci * _C_CH, _C_CH)])
        pltpu.async_copy(src_hbm.at[i1_v.at[sl]], buf, sem).wait()
        pltpu.sync_copy(buf, g1_hbm.at[pl.ds(base + ci * _C_CH, _C_CH)])


@functools.lru_cache(maxsize=None)
def _combine_kernel():
    return pl.kernel(
        _combine_body,
        out_type=(
            jax.ShapeDtypeStruct((N, C), jnp.float32),
            jax.ShapeDtypeStruct((N, C), jnp.float32),
        ),
        mesh=_sc_mesh(),
        scratch_types=[
            pltpu.VMEM((_C_RPW,), jnp.int32),
            pltpu.VMEM((_C_RPW,), jnp.int32),
            pltpu.VMEM((_C_CH, C), jnp.float32),
            pltpu.SemaphoreType.DMA,
        ],
    )


def _combine(out_perm, pos0, pos1):
    return _combine_kernel()(out_perm, pos0, pos1)


def _shared_body(x_ref, sg_ref, sd_ref, g0_ref, g1_ref, o_ref):
    yg = jnp.dot(x_ref[...], sg_ref[...], preferred_element_type=jnp.float32)
    y = yg[:, :HS]
    gate = yg[:, HS:]
    h = gate * jax.nn.sigmoid(gate) * y
    o_ref[...] = (jnp.dot(h, sd_ref[...], preferred_element_type=jnp.float32)
                  + g0_ref[...] + g1_ref[...])


def _shared(x_flat, sgw, sdw, g0, g1):
    tm = 128
    return pl.pallas_call(
        _shared_body,
        grid=(N // tm,),
        in_specs=[
            pl.BlockSpec((tm, C), lambda t: (t, 0)),
            pl.BlockSpec((C, 2 * HS), lambda t: (0, 0)),
            pl.BlockSpec((HS, C), lambda t: (0, 0)),
            pl.BlockSpec((tm, C), lambda t: (t, 0)),
            pl.BlockSpec((tm, C), lambda t: (t, 0)),
        ],
        out_specs=pl.BlockSpec((tm, C), lambda t: (t, 0)),
        out_shape=jax.ShapeDtypeStruct((N, C), jnp.float32),
    )(x_flat, sgw, sdw, g0, g1)


def kernel(x, gate_w, gate_bias, shared_gate_w, shared_down_w, up_w, down_w):
    x_flat = x.reshape(N, C)

    idx_out, w_out = _router(x_flat, gate_w, gate_bias)
    eN = idx_out[:, :TOPK].reshape(-1)
    wN = w_out[:, :TOPK].reshape(-1)

    onehot = (eN[:, None] == jnp.arange(E, dtype=jnp.int32)[None, :])
    ranks = jnp.cumsum(onehot.astype(jnp.int32), axis=0)
    counts = ranks[-1]
    rank = jnp.take_along_axis(ranks, eN[:, None], axis=1)[:, 0] - 1
    padded = ((counts + TM - 1) // TM) * TM
    pad_off = jnp.concatenate(
        [jnp.zeros((1,), jnp.int32), jnp.cumsum(padded).astype(jnp.int32)])
    ppos = pad_off[eN] + rank
    tok = (jnp.arange(NA, dtype=jnp.int32) // TOPK)
    tok_pad = jnp.zeros((P,), jnp.int32).at[ppos].set(tok)
    w_pad = jnp.zeros((P,), jnp.float32).at[ppos].set(wN)
    wpad2d = jnp.broadcast_to(w_pad[:, None], (P, 128))
    tile_off = pad_off // TM
    t_ar = jnp.arange(W, dtype=jnp.int32)
    e_for_tile = jnp.minimum(
        jnp.sum((t_ar[:, None] >= tile_off[None, 1:]).astype(jnp.int32),
                axis=1), E - 1).astype(jnp.int32)
    valid = (t_ar < tile_off[E]).astype(jnp.int32)
    pos0 = ppos[0::2]
    pos1 = ppos[1::2]

    x_perm = _dispatch(x_flat, tok_pad)

    out_perm = _grouped_gemm(x_perm, up_w, down_w, wpad2d, e_for_tile, valid)

    g0, g1 = _combine(out_perm, pos0, pos1)

    out = _shared(x_flat, shared_gate_w, shared_down_w, g0, g1)
    return out.reshape(B, T, C)
